# baseline (speedup 1.0000x reference)
import numpy as np

import jax
import jax.numpy as jnp
from jax import lax
from jax.experimental import pallas as pl
from jax.experimental.pallas import tpu as pltpu


def _body_fused(offs_ref, xt_ref, wt_ref, s_ref, o_ref, gm_ref):
    @pl.when(pl.program_id(0) == 0)
    def _build():
        V = wt_ref.shape[1]
        D = offs_ref.shape[0]
        oh = (lax.broadcasted_iota(jnp.int32, (D, V), 1)
              == offs_ref[...]).astype(jnp.bfloat16)
        m = lax.dot_general(
            oh, wt_ref[...].astype(jnp.bfloat16),
            dimension_numbers=(((1,), (1,)), ((), ())),
            preferred_element_type=jnp.float32,
        )
        rep = pltpu.repeat(m.astype(jnp.bfloat16), D, axis=1)
        gm_ref[...] = rep * s_ref[...]

    xb = xt_ref[...].astype(jnp.bfloat16)
    o_ref[...] = lax.dot_general(
        xb, gm_ref[...],
        dimension_numbers=(((0,), (0,)), ((), ())),
        preferred_element_type=jnp.float32,
    )


def kernel(x, weight, offsets):
    B, D = x.shape
    V, E = weight.shape
    DE = D * E

    sel = (np.arange(DE, dtype=np.int32)[None, :] // E
           == np.arange(D, dtype=np.int32)[:, None])
    s = jnp.asarray(sel.astype(np.float32), dtype=jnp.bfloat16)

    tb = 2048
    if B % tb != 0:
        tb = max(8, min(tb, B))
    grid = (pl.cdiv(B, tb),)

    return pl.pallas_call(
        _body_fused,
        out_shape=jax.ShapeDtypeStruct((B, DE), jnp.float32),
        grid=grid,
        in_specs=[
            pl.BlockSpec((D, 1), lambda i: (0, 0)),
            pl.BlockSpec((D, tb), lambda i: (0, i)),
            pl.BlockSpec((E, V), lambda i: (0, 0)),
            pl.BlockSpec((D, DE), lambda i: (0, 0)),
        ],
        out_specs=pl.BlockSpec((tb, DE), lambda i: (i, 0)),
        scratch_shapes=[pltpu.VMEM((D, DE), jnp.bfloat16)],
        compiler_params=pltpu.CompilerParams(
            dimension_semantics=("arbitrary",),
        ),
        cost_estimate=pl.CostEstimate(
            flops=2 * B * D * DE,
            transcendentals=0,
            bytes_accessed=4 * (B * DE + B * D) + 4 * E * V + 2 * D * DE,
        ),
    )(offsets.reshape(D, 1), x.T, weight.T, s)


# speedup vs baseline: 1.0936x; 1.0936x over previous
import numpy as np

import jax
import jax.numpy as jnp
from jax import lax
from jax.experimental import pallas as pl
from jax.experimental.pallas import tpu as pltpu


def _body_fused(offs_ref, xt_ref, wt_ref, s_ref, o_ref, gm_ref):
    @pl.when(pl.program_id(0) == 0)
    def _build():
        V = wt_ref.shape[1]
        D = offs_ref.shape[0]
        oh = (lax.broadcasted_iota(jnp.int32, (D, V), 1)
              == offs_ref[...]).astype(jnp.bfloat16)
        m = lax.dot_general(
            oh, wt_ref[...].astype(jnp.bfloat16),
            dimension_numbers=(((1,), (1,)), ((), ())),
            preferred_element_type=jnp.float32,
        )
        rep = pltpu.repeat(m.astype(jnp.bfloat16), D, axis=1)
        gm_ref[...] = rep * s_ref[...]

    xb = xt_ref[...].astype(jnp.bfloat16)
    o_ref[...] = lax.dot_general(
        xb, gm_ref[...],
        dimension_numbers=(((0,), (0,)), ((), ())),
        preferred_element_type=jnp.float32,
    )


def kernel(x, weight, offsets):
    B, D = x.shape
    V, E = weight.shape
    DE = D * E

    sel = (np.arange(DE, dtype=np.int32)[None, :] // E
           == np.arange(D, dtype=np.int32)[:, None])
    s = jnp.asarray(sel.astype(np.float32), dtype=jnp.bfloat16)

    tb = 4096
    if B % tb != 0:
        tb = max(8, min(tb, B))
    grid = (pl.cdiv(B, tb),)

    return pl.pallas_call(
        _body_fused,
        out_shape=jax.ShapeDtypeStruct((B, DE), jnp.float32),
        grid=grid,
        in_specs=[
            pl.BlockSpec((D, 1), lambda i: (0, 0)),
            pl.BlockSpec((D, tb), lambda i: (0, i)),
            pl.BlockSpec((E, V), lambda i: (0, 0)),
            pl.BlockSpec((D, DE), lambda i: (0, 0)),
        ],
        out_specs=pl.BlockSpec((tb, DE), lambda i: (i, 0)),
        scratch_shapes=[pltpu.VMEM((D, DE), jnp.bfloat16)],
        compiler_params=pltpu.CompilerParams(
            dimension_semantics=("arbitrary",),
        ),
        cost_estimate=pl.CostEstimate(
            flops=2 * B * D * DE,
            transcendentals=0,
            bytes_accessed=4 * (B * DE + B * D) + 4 * E * V + 2 * D * DE,
        ),
    )(offsets.reshape(D, 1), x.T, weight.T, s)


# speedup vs baseline: 1.0971x; 1.0033x over previous
import numpy as np

import jax
import jax.numpy as jnp
from jax import lax
from jax.experimental import pallas as pl
from jax.experimental.pallas import tpu as pltpu


def _body(offs_ref, xt_ref, wt_ref, s_ref, o_ref, gm_ref, oh_ref):
    @pl.when(pl.program_id(0) == 0)
    def _build():
        D = s_ref.shape[0]
        V = wt_ref.shape[1]
        lane = lax.broadcasted_iota(jnp.int32, (1, V), 1)
        for d in range(D):
            oh_ref[d:d + 1, :] = (lane == offs_ref[d]).astype(jnp.bfloat16)
        m = lax.dot_general(
            oh_ref[...], wt_ref[...].astype(jnp.bfloat16),
            dimension_numbers=(((1,), (1,)), ((), ())),
            preferred_element_type=jnp.float32,
        )
        rep = pltpu.repeat(m.astype(jnp.bfloat16), D, axis=1)
        gm_ref[...] = rep * s_ref[...]

    xb = xt_ref[...].astype(jnp.bfloat16)
    o_ref[...] = lax.dot_general(
        xb, gm_ref[...],
        dimension_numbers=(((0,), (0,)), ((), ())),
        preferred_element_type=jnp.float32,
    )


def kernel(x, weight, offsets):
    B, D = x.shape
    V, E = weight.shape
    DE = D * E

    sel = (np.arange(DE, dtype=np.int32)[None, :] // E
           == np.arange(D, dtype=np.int32)[:, None])
    s = jnp.asarray(sel.astype(np.float32), dtype=jnp.bfloat16)

    tb = 4096
    if B % tb != 0:
        tb = max(8, min(tb, B))
    grid = (pl.cdiv(B, tb),)

    return pl.pallas_call(
        _body,
        out_shape=jax.ShapeDtypeStruct((B, DE), jnp.float32),
        grid_spec=pltpu.PrefetchScalarGridSpec(
            num_scalar_prefetch=1,
            grid=grid,
            in_specs=[
                pl.BlockSpec((D, tb), lambda i, offs: (0, i)),
                pl.BlockSpec((E, V), lambda i, offs: (0, 0)),
                pl.BlockSpec((D, DE), lambda i, offs: (0, 0)),
            ],
            out_specs=pl.BlockSpec((tb, DE), lambda i, offs: (i, 0)),
            scratch_shapes=[pltpu.VMEM((D, DE), jnp.bfloat16),
                            pltpu.VMEM((D, V), jnp.bfloat16)],
        ),
        compiler_params=pltpu.CompilerParams(
            dimension_semantics=("arbitrary",),
        ),
        cost_estimate=pl.CostEstimate(
            flops=2 * B * D * DE,
            transcendentals=0,
            bytes_accessed=4 * (B * DE + B * D) + 4 * E * V + 2 * D * DE,
        ),
    )(offsets, x.T, weight.T, s)


# speedup vs baseline: 1.0980x; 1.0008x over previous
import jax
import jax.numpy as jnp
from jax import lax
from jax.experimental import pallas as pl
from jax.experimental.pallas import tpu as pltpu


def _body(offs_ref, xt_ref, wt_ref, o_ref, gm_ref, oh_ref):
    @pl.when(pl.program_id(0) == 0)
    def _build():
        D, DE = gm_ref.shape
        V = wt_ref.shape[1]
        E = DE // D
        lane = lax.broadcasted_iota(jnp.int32, (1, V), 1)
        for d in range(D):
            oh_ref[d:d + 1, :] = (lane == offs_ref[d]).astype(jnp.bfloat16)
        m = lax.dot_general(
            oh_ref[...], wt_ref[...].astype(jnp.bfloat16),
            dimension_numbers=(((1,), (1,)), ((), ())),
            preferred_element_type=jnp.float32,
        )
        rep = pltpu.repeat(m.astype(jnp.bfloat16), D, axis=1)
        sel = (lax.broadcasted_iota(jnp.int32, (D, DE), 1) // E
               == lax.broadcasted_iota(jnp.int32, (D, DE), 0))
        gm_ref[...] = jnp.where(sel, rep, jnp.bfloat16(0.0))

    xb = xt_ref[...].astype(jnp.bfloat16)
    o_ref[...] = lax.dot_general(
        xb, gm_ref[...],
        dimension_numbers=(((0,), (0,)), ((), ())),
        preferred_element_type=jnp.float32,
    )


def kernel(x, weight, offsets):
    B, D = x.shape
    V, E = weight.shape
    DE = D * E

    tb = 4096
    if B % tb != 0:
        tb = max(8, min(tb, B))
    grid = (pl.cdiv(B, tb),)

    return pl.pallas_call(
        _body,
        out_shape=jax.ShapeDtypeStruct((B, DE), jnp.float32),
        grid_spec=pltpu.PrefetchScalarGridSpec(
            num_scalar_prefetch=1,
            grid=grid,
            in_specs=[
                pl.BlockSpec((D, tb), lambda i, offs: (0, i)),
                pl.BlockSpec((E, V), lambda i, offs: (0, 0)),
            ],
            out_specs=pl.BlockSpec((tb, DE), lambda i, offs: (i, 0)),
            scratch_shapes=[pltpu.VMEM((D, DE), jnp.bfloat16),
                            pltpu.VMEM((D, V), jnp.bfloat16)],
        ),
        compiler_params=pltpu.CompilerParams(
            dimension_semantics=("arbitrary",),
        ),
        cost_estimate=pl.CostEstimate(
            flops=2 * B * D * DE,
            transcendentals=0,
            bytes_accessed=4 * (B * DE + B * D) + 4 * E * V,
        ),
    )(offsets, x.T, weight.T)
